# baseline (speedup 1.0000x reference)
import functools

import jax
import jax.numpy as jnp
from jax import lax
from jax.experimental import pallas as pl
from jax.experimental.pallas import tpu as pltpu
from jax.experimental.pallas import tpu_sc as plsc

_EMBED_DIM = 64
_HIDDEN = 128
_TAGS = 128

_NUM_WORKERS = 32
_GATHER_WIN = 320
_TBL_BLK = 8192


def _sc_gather(table, idx_flat):
    n = idx_flat.shape[0]
    d = table.shape[1]
    per_w = n // _NUM_WORKERS
    nch = per_w // _GATHER_WIN
    w = _GATHER_WIN
    mesh = plsc.VectorSubcoreMesh(core_axis_name="core", subcore_axis_name="subcore")

    @functools.partial(
        pl.kernel,
        out_type=jax.ShapeDtypeStruct((n, d), table.dtype),
        mesh=mesh,
        scratch_types=[
            pltpu.VMEM((per_w,), jnp.int32),
            pltpu.VMEM((w, d), table.dtype),
            pltpu.VMEM((w, d), table.dtype),
            pltpu.SemaphoreType.DMA,
            pltpu.SemaphoreType.DMA,
            pltpu.SemaphoreType.DMA,
            pltpu.SemaphoreType.DMA,
        ],
    )
    def gather_kernel(tbl_hbm, idx_hbm, out_hbm, idx_v, buf0, buf1,
                      gs0, gs1, os0, os1):
        wid = lax.axis_index("subcore") * 2 + lax.axis_index("core")
        base = wid * per_w
        pltpu.sync_copy(idx_hbm.at[pl.ds(base, per_w)], idx_v)

        def gth(c, buf, sem):
            return pltpu.make_async_copy(
                tbl_hbm.at[idx_v.at[pl.ds(c * w, w)]], buf, sem)

        def put(c, buf, sem):
            return pltpu.make_async_copy(
                buf, out_hbm.at[pl.ds(base + c * w, w)], sem)

        gth(0, buf0, gs0).start()

        @pl.loop(0, nch, step=2)
        def _(c):
            @pl.when(c > 0)
            def _():
                put(c - 1, buf1, os1).wait()

            gth(c + 1, buf1, gs1).start()
            gth(c, buf0, gs0).wait()
            put(c, buf0, os0).start()
            gth(c + 1, buf1, gs1).wait()
            put(c + 1, buf1, os1).start()

            @pl.when(c + 2 < nch)
            def _():
                put(c, buf0, os0).wait()
                gth(c + 2, buf0, gs0).start()

        put(nch - 2, buf0, os0).wait()
        put(nch - 1, buf1, os1).wait()

    return gather_kernel(table, idx_flat)


def _mlp_body(et_ref, w1_ref, b1_ref, w2_ref, b2_ref, w3_ref, b3_ref, o_ref):
    h = jax.lax.dot_general(
        et_ref[...].astype(jnp.bfloat16), w1_ref[...].astype(jnp.bfloat16),
        (((0,), (0,)), ((), ())),
        preferred_element_type=jnp.float32,
    )
    h = jnp.maximum(h + b1_ref[...], 0.0)
    h = jnp.dot(h.astype(jnp.bfloat16), w2_ref[...].astype(jnp.bfloat16),
                preferred_element_type=jnp.float32)
    h = jnp.maximum(h + b2_ref[...], 0.0)
    o_ref[...] = jnp.dot(h.astype(jnp.bfloat16), w3_ref[...].astype(jnp.bfloat16),
                         preferred_element_type=jnp.float32) + b3_ref[...]


def _tc_table_mlp(embt, W1, b1, W2, b2, W3, b3):
    v = embt.shape[1]
    grid = pl.cdiv(v, _TBL_BLK)
    v_pad = grid * _TBL_BLK
    return pl.pallas_call(
        _mlp_body,
        grid=(grid,),
        in_specs=[
            pl.BlockSpec((_EMBED_DIM, _TBL_BLK), lambda i: (0, i)),
            pl.BlockSpec((_EMBED_DIM, _HIDDEN), lambda i: (0, 0)),
            pl.BlockSpec((1, _HIDDEN), lambda i: (0, 0)),
            pl.BlockSpec((_HIDDEN, _HIDDEN), lambda i: (0, 0)),
            pl.BlockSpec((1, _HIDDEN), lambda i: (0, 0)),
            pl.BlockSpec((_HIDDEN, _TAGS), lambda i: (0, 0)),
            pl.BlockSpec((1, _TAGS), lambda i: (0, 0)),
        ],
        out_specs=pl.BlockSpec((_TBL_BLK, _TAGS), lambda i: (i, 0)),
        out_shape=jax.ShapeDtypeStruct((v_pad, _TAGS), jnp.float32),
    )(embt, W1, b1.reshape(1, -1), W2, b2.reshape(1, -1), W3, b3.reshape(1, -1))


def kernel(x, emb, W1, b1, W2, b2, W3, b3):
    b, l = x.shape
    xt = x.astype(jnp.int32).T.reshape(-1)
    table = _tc_table_mlp(emb.T, W1, b1, W2, b2, W3, b3)
    out = _sc_gather(table, xt)
    return out.reshape(l, b, _TAGS).transpose(1, 0, 2)


# speedup vs baseline: 1.0240x; 1.0240x over previous
import functools

import jax
import jax.numpy as jnp
from jax import lax
from jax.experimental import pallas as pl
from jax.experimental.pallas import tpu as pltpu
from jax.experimental.pallas import tpu_sc as plsc

_EMBED_DIM = 64
_HIDDEN = 128
_TAGS = 128

_GATHER_WINDOW = 256
_TBL_BLK = 8192


def _sc_gather(table, xt3):
    nblk, _, w = xt3.shape
    n = nblk * w
    d = table.shape[1]
    mesh = plsc.VectorSubcoreMesh(core_axis_name="core", subcore_axis_name="subcore")

    @functools.partial(
        pl.kernel,
        out_type=jax.ShapeDtypeStruct((n, d), table.dtype),
        mesh=mesh,
    )
    def gather_kernel(tbl_hbm, idx_hbm, out_hbm):
        def body(idx_vmem, out_vmem):
            pltpu.sync_copy(tbl_hbm.at[idx_vmem.at[0, 0]], out_vmem)

        pltpu.emit_pipeline(
            body,
            grid=(nblk,),
            in_specs=[pl.BlockSpec((1, 1, w), lambda i: (i, 0, 0))],
            out_specs=[pl.BlockSpec((w, d), lambda i: (i, 0))],
            core_axis_name=("core", "subcore"),
            dimension_semantics=(pltpu.PARALLEL,),
        )(idx_hbm, out_hbm)

    return gather_kernel(table, xt3)


def _mlp_body(et_ref, w1_ref, b1_ref, w2_ref, b2_ref, w3_ref, b3_ref, o_ref):
    h = jax.lax.dot_general(
        et_ref[...].astype(jnp.bfloat16), w1_ref[...].astype(jnp.bfloat16),
        (((0,), (0,)), ((), ())),
        preferred_element_type=jnp.float32,
    )
    h = jnp.maximum(h + b1_ref[...], 0.0)
    h = jnp.dot(h.astype(jnp.bfloat16), w2_ref[...].astype(jnp.bfloat16),
                preferred_element_type=jnp.float32)
    h = jnp.maximum(h + b2_ref[...], 0.0)
    o_ref[...] = jnp.dot(h.astype(jnp.bfloat16), w3_ref[...].astype(jnp.bfloat16),
                         preferred_element_type=jnp.float32) + b3_ref[...]


def _tc_table_mlp(embt, W1, b1, W2, b2, W3, b3):
    v = embt.shape[1]
    grid = pl.cdiv(v, _TBL_BLK)
    v_pad = grid * _TBL_BLK
    return pl.pallas_call(
        _mlp_body,
        grid=(grid,),
        in_specs=[
            pl.BlockSpec((_EMBED_DIM, _TBL_BLK), lambda i: (0, i)),
            pl.BlockSpec((_EMBED_DIM, _HIDDEN), lambda i: (0, 0)),
            pl.BlockSpec((1, _HIDDEN), lambda i: (0, 0)),
            pl.BlockSpec((_HIDDEN, _HIDDEN), lambda i: (0, 0)),
            pl.BlockSpec((1, _HIDDEN), lambda i: (0, 0)),
            pl.BlockSpec((_HIDDEN, _TAGS), lambda i: (0, 0)),
            pl.BlockSpec((1, _TAGS), lambda i: (0, 0)),
        ],
        out_specs=pl.BlockSpec((_TBL_BLK, _TAGS), lambda i: (i, 0)),
        out_shape=jax.ShapeDtypeStruct((v_pad, _TAGS), jnp.float32),
    )(embt, W1, b1.reshape(1, -1), W2, b2.reshape(1, -1), W3, b3.reshape(1, -1))


def kernel(x, emb, W1, b1, W2, b2, W3, b3):
    b, l = x.shape
    w = _GATHER_WINDOW
    xt3 = x.astype(jnp.int32).T.reshape(l * b // w, 1, w)
    table = _tc_table_mlp(emb.T, W1, b1, W2, b2, W3, b3)
    out = _sc_gather(table, xt3)
    return out.reshape(l, b, _TAGS).transpose(1, 0, 2)


# speedup vs baseline: 1.0294x; 1.0052x over previous
import functools

import jax
import jax.numpy as jnp
from jax import lax
from jax.experimental import pallas as pl
from jax.experimental.pallas import tpu as pltpu
from jax.experimental.pallas import tpu_sc as plsc

_EMBED_DIM = 64
_HIDDEN = 128
_TAGS = 128

_GATHER_WINDOW = 256
_TBL_BLK = 16384


def _sc_gather(table, xt3):
    nblk, _, w = xt3.shape
    n = nblk * w
    d = table.shape[1]
    mesh = plsc.VectorSubcoreMesh(core_axis_name="core", subcore_axis_name="subcore")

    @functools.partial(
        pl.kernel,
        out_type=jax.ShapeDtypeStruct((n, d), table.dtype),
        mesh=mesh,
    )
    def gather_kernel(tbl_hbm, idx_hbm, out_hbm):
        def body(idx_vmem, out_vmem):
            pltpu.sync_copy(tbl_hbm.at[idx_vmem.at[0, 0]], out_vmem)

        pltpu.emit_pipeline(
            body,
            grid=(nblk,),
            in_specs=[pl.BlockSpec((1, 1, w), lambda i: (i, 0, 0))],
            out_specs=[pl.BlockSpec((w, d), lambda i: (i, 0))],
            core_axis_name=("core", "subcore"),
            dimension_semantics=(pltpu.PARALLEL,),
        )(idx_hbm, out_hbm)

    return gather_kernel(table, xt3)


def _mlp_body(et_ref, w1_ref, b1_ref, w2_ref, b2_ref, w3_ref, b3_ref, o_ref):
    h = jax.lax.dot_general(
        et_ref[...].astype(jnp.bfloat16), w1_ref[...].astype(jnp.bfloat16),
        (((0,), (0,)), ((), ())),
        preferred_element_type=jnp.float32,
    )
    h = jnp.maximum(h + b1_ref[...], 0.0)
    h = jnp.dot(h.astype(jnp.bfloat16), w2_ref[...].astype(jnp.bfloat16),
                preferred_element_type=jnp.float32)
    h = jnp.maximum(h + b2_ref[...], 0.0)
    o_ref[...] = jnp.dot(h.astype(jnp.bfloat16), w3_ref[...].astype(jnp.bfloat16),
                         preferred_element_type=jnp.float32) + b3_ref[...]


def _tc_table_mlp(embt, W1, b1, W2, b2, W3, b3):
    v = embt.shape[1]
    grid = pl.cdiv(v, _TBL_BLK)
    v_pad = grid * _TBL_BLK
    return pl.pallas_call(
        _mlp_body,
        grid=(grid,),
        in_specs=[
            pl.BlockSpec((_EMBED_DIM, _TBL_BLK), lambda i: (0, i)),
            pl.BlockSpec((_EMBED_DIM, _HIDDEN), lambda i: (0, 0)),
            pl.BlockSpec((1, _HIDDEN), lambda i: (0, 0)),
            pl.BlockSpec((_HIDDEN, _HIDDEN), lambda i: (0, 0)),
            pl.BlockSpec((1, _HIDDEN), lambda i: (0, 0)),
            pl.BlockSpec((_HIDDEN, _TAGS), lambda i: (0, 0)),
            pl.BlockSpec((1, _TAGS), lambda i: (0, 0)),
        ],
        out_specs=pl.BlockSpec((_TBL_BLK, _TAGS), lambda i: (i, 0)),
        out_shape=jax.ShapeDtypeStruct((v_pad, _TAGS), jnp.float32),
    )(embt, W1, b1.reshape(1, -1), W2, b2.reshape(1, -1), W3, b3.reshape(1, -1))


def kernel(x, emb, W1, b1, W2, b2, W3, b3):
    b, l = x.shape
    w = _GATHER_WINDOW
    xt3 = x.astype(jnp.int32).T.reshape(l * b // w, 1, w)
    table = _tc_table_mlp(emb.T, W1, b1, W2, b2, W3, b3)
    out = _sc_gather(table, xt3)
    return out.reshape(l, b, _TAGS).transpose(1, 0, 2)


# speedup vs baseline: 1.0475x; 1.0176x over previous
import functools

import jax
import jax.numpy as jnp
from jax import lax
from jax.experimental import pallas as pl
from jax.experimental.pallas import tpu as pltpu
from jax.experimental.pallas import tpu_sc as plsc

_EMBED_DIM = 64
_HIDDEN = 128
_TAGS = 128

_GATHER_WINDOW = 256
_TBL_BLK = 16384


def _sc_gather(table, xt3):
    nblk, _, w = xt3.shape
    n = nblk * w
    d = table.shape[1]
    mesh = plsc.VectorSubcoreMesh(core_axis_name="core", subcore_axis_name="subcore")

    @functools.partial(
        pl.kernel,
        out_type=jax.ShapeDtypeStruct((n, d), table.dtype),
        mesh=mesh,
    )
    def gather_kernel(tbl_hbm, idx_hbm, out_hbm):
        def body(idx_vmem, out_vmem):
            pltpu.sync_copy(tbl_hbm.at[idx_vmem.at[0, 0]], out_vmem)

        pltpu.emit_pipeline(
            body,
            grid=(nblk,),
            in_specs=[pl.BlockSpec((1, 1, w), lambda i: (i, 0, 0))],
            out_specs=[pl.BlockSpec((w, d), lambda i: (i, 0))],
            core_axis_name=("core", "subcore"),
            dimension_semantics=(pltpu.PARALLEL,),
        )(idx_hbm, out_hbm)

    return gather_kernel(table, xt3)


def _mlp_body(et_ref, w1_ref, b1_ref, w2_ref, b2_ref, w3_ref, b3_ref, o_ref):
    h = jax.lax.dot_general(
        et_ref[...].astype(jnp.bfloat16), w1_ref[...].astype(jnp.bfloat16),
        (((0,), (0,)), ((), ())),
        preferred_element_type=jnp.float32,
    )
    h = jnp.maximum(h + b1_ref[...], 0.0)
    h = jnp.dot(h.astype(jnp.bfloat16), w2_ref[...].astype(jnp.bfloat16),
                preferred_element_type=jnp.float32)
    h = jnp.maximum(h + b2_ref[...], 0.0)
    o_ref[...] = jnp.dot(h.astype(jnp.bfloat16), w3_ref[...].astype(jnp.bfloat16),
                         preferred_element_type=jnp.float32) + b3_ref[...]


def _tc_table_mlp(embt, W1, b1, W2, b2, W3, b3):
    v = embt.shape[1]
    grid = pl.cdiv(v, _TBL_BLK)
    return pl.pallas_call(
        _mlp_body,
        grid=(grid,),
        in_specs=[
            pl.BlockSpec((_EMBED_DIM, _TBL_BLK), lambda i: (0, i)),
            pl.BlockSpec((_EMBED_DIM, _HIDDEN), lambda i: (0, 0)),
            pl.BlockSpec((1, _HIDDEN), lambda i: (0, 0)),
            pl.BlockSpec((_HIDDEN, _HIDDEN), lambda i: (0, 0)),
            pl.BlockSpec((1, _HIDDEN), lambda i: (0, 0)),
            pl.BlockSpec((_HIDDEN, _TAGS), lambda i: (0, 0)),
            pl.BlockSpec((1, _TAGS), lambda i: (0, 0)),
        ],
        out_specs=pl.BlockSpec((_TBL_BLK, _TAGS), lambda i: (i, 0)),
        out_shape=jax.ShapeDtypeStruct((v, _TAGS), jnp.float32),
    )(embt, W1, b1.reshape(1, -1), W2, b2.reshape(1, -1), W3, b3.reshape(1, -1))


def kernel(x, emb, W1, b1, W2, b2, W3, b3):
    b, l = x.shape
    w = _GATHER_WINDOW
    xt3 = x.astype(jnp.int32).T.reshape(l * b // w, 1, w)
    table = _tc_table_mlp(emb.T, W1, b1, W2, b2, W3, b3)
    out = _sc_gather(table, xt3)
    return out.reshape(l, b, _TAGS).transpose(1, 0, 2)
